# baseline (speedup 1.0000x reference)
import functools

import jax
import jax.numpy as jnp
from jax import lax
from jax.experimental import pallas as pl
from jax.experimental.pallas import tpu as pltpu
from jax.experimental.pallas import tpu_sc as plsc

N_PATCHES = 200
AH = AW = 128
H = W = 2048
C = 3
PATCH = 32
SY = H // AH
NC, NS = 2, 16
NW = NC * NS
UNITS = 2 * N_PATCHES * C
UPW = 38
UPAD = NW * UPW
DROWS = 40
BLKW = 256
KPAD = 256


def _topk_body(score_ref, p_ref, sa_ref, idx_ref, s_ref):
    s_ref[...] = score_ref[...]
    lane = lax.broadcasted_iota(jnp.int32, (KPAD,), 0)
    io_r = lax.broadcasted_iota(jnp.int32, (AH, 1), 0)
    io_c = lax.broadcasted_iota(jnp.int32, (1, AW), 1)
    big = jnp.int32(1 << 30)

    def rowfold(s):
        m = s
        w = AW
        while w > 1:
            w //= 2
            m = jnp.maximum(m[:, :w], m[:, w:])
        return m

    def step(j, b, rowmax, idxv, sav):
        m = jnp.max(rowmax)
        y = jnp.min(jnp.where(rowmax == m, io_r, big))
        row = s_ref[b, pl.ds(y, 1), :]
        x = jnp.min(jnp.where(row == m, io_c, big))
        prow = p_ref[b, pl.ds(y, 1), :]
        hit = io_c == x
        pv = jnp.sum(jnp.where(hit, prow, jnp.float32(0.0)))
        newrow = jnp.where(hit, jnp.float32(-1e30), row)
        s_ref[b, pl.ds(y, 1), :] = newrow
        rowmax = jnp.where(io_r == y, jnp.max(newrow), rowmax)
        idxv = jnp.where(lane == j, y * AW + x, idxv)
        sav = jnp.where(lane == j, pv, sav)
        return rowmax, idxv, sav

    def body(j, st):
        r0, r1, i0, i1, a0, a1 = st
        r0, i0, a0 = step(j, 0, r0, i0, a0)
        r1, i1, a1 = step(j, 1, r1, i1, a1)
        return r0, r1, i0, i1, a0, a1

    z_i = jnp.zeros((KPAD,), jnp.int32)
    z_f = jnp.zeros((KPAD,), jnp.float32)
    _, _, i0, i1, a0, a1 = lax.fori_loop(
        0, N_PATCHES, body,
        (rowfold(score_ref[0]), rowfold(score_ref[1]),
         z_i, z_i, z_f, z_f))

    idx_ref[0, 0] = i0
    idx_ref[1, 0] = i1
    sa_ref[0, 0] = a0
    sa_ref[1, 0] = a1


def _topk_call(score, p):
    return pl.pallas_call(
        _topk_body,
        out_shape=[jax.ShapeDtypeStruct((2, 1, KPAD), jnp.float32),
                   jax.ShapeDtypeStruct((2, 1, KPAD), jnp.int32)],
        scratch_shapes=[pltpu.VMEM((2, AH, AW), jnp.float32)],
    )(score, p)


@functools.cache
def _make_gather():
    mesh = plsc.VectorSubcoreMesh(core_axis_name="c", subcore_axis_name="s")

    @functools.partial(
        pl.kernel,
        mesh=mesh,
        out_type=jax.ShapeDtypeStruct((UPAD, PATCH, PATCH), jnp.float32),
        compiler_params=pltpu.CompilerParams(use_tc_tiling_on_sc=True),
        scratch_types=[
            pltpu.VMEM((DROWS, 128), jnp.int32),
            pltpu.VMEM((PATCH, BLKW), jnp.float32),
            pltpu.VMEM((PATCH, BLKW), jnp.float32),
            pltpu.VMEM((PATCH, PATCH), jnp.float32),
            pltpu.VMEM((PATCH, PATCH), jnp.float32),
            pltpu.SemaphoreType.DMA,
            pltpu.SemaphoreType.DMA,
            pltpu.SemaphoreType.DMA,
            pltpu.SemaphoreType.DMA,
        ],
    )
    def gather_k(wsi_hbm, desc_hbm, out_hbm, desc_v, buf0, buf1,
                 pbuf0, pbuf1, sr0, sr1, sw0, sw1):
        wid = lax.axis_index("s") * NC + lax.axis_index("c")
        pltpu.sync_copy(desc_hbm.at[wid], desc_v)
        lane16 = lax.broadcasted_iota(jnp.int32, (16,), 0)
        bufs = (buf0, buf1)
        pbufs = (pbuf0, pbuf1)
        srs = (sr0, sr1)
        sws = (sw0, sw1)

        def fields(t):
            v = desc_v[t, pl.ds(0, 16)]
            return v[0], v[1], v[2], v[3]

        def start_read(t, buf, sem):
            row0, xa, _, _ = fields(t)
            row0 = pl.multiple_of(row0, 16)
            xa = pl.multiple_of(xa, 128)
            return pltpu.async_copy(
                wsi_hbm.at[pl.ds(row0, PATCH), pl.ds(xa, BLKW)], buf, sem)

        reads = [start_read(0, buf0, sr0), start_read(1, buf1, sr1)]
        writes = [None, None]
        for t in range(UPW):
            pipe = t % 2
            buf = bufs[pipe]
            pbuf = pbufs[pipe]
            reads[pipe].wait()
            if writes[pipe] is not None:
                writes[pipe].wait()
            _, _, xoff, uout = fields(t)
            xoff = pl.multiple_of(xoff, 16)
            for r in range(PATCH):
                for h in range(2):
                    pbuf[r, pl.ds(h * 16, 16)] = (
                        buf[r, pl.ds(xoff + h * 16, 16)])
            writes[pipe] = pltpu.async_copy(
                pbuf, out_hbm.at[uout], sws[pipe])
            if t + 2 < UPW:
                reads[pipe] = start_read(t + 2, buf, srs[pipe])
        writes[0].wait()
        writes[1].wait()

    return gather_k


def kernel(x_low, x_high, attention, WSI):
    B = attention.shape[0]
    flat = attention.reshape(B, -1)
    p = flat / jnp.sum(flat, axis=-1, keepdims=True)
    logp = jnp.log(p + 1e-12)
    u = jax.random.uniform(jax.random.key(42), flat.shape,
                           minval=1e-9, maxval=1.0)
    gumbel = -jnp.log(-jnp.log(u))
    score = logp + gumbel
    sa_pad, idx_pad = _topk_call(score.reshape(B, AH, AW),
                                 p.reshape(B, AH, AW))

    idx_flat = idx_pad.reshape(B, KPAD)
    uu = jnp.arange(UPAD, dtype=jnp.int32)
    bb = jnp.minimum(uu // (N_PATCHES * C), B - 1)
    nn = (uu % (N_PATCHES * C)) // C
    cc = uu % C
    cell = idx_flat[bb, nn]
    ys = cell // AW
    xs = cell % AW
    y0 = jnp.minimum(ys * SY, H - PATCH)
    x0 = jnp.minimum(xs * SY, W - PATCH)
    xa = jnp.minimum((x0 // 128) * 128, W - BLKW)
    xoff = x0 - xa
    row0 = (bb * C + cc) * H + y0
    fields = jnp.stack([row0, xa, xoff, uu], axis=-1).astype(jnp.int32)
    desc = jnp.zeros((NW, DROWS, 128), jnp.int32)
    desc = desc.at[:, :UPW, :4].set(fields.reshape(NW, UPW, 4))

    out3 = _make_gather()(WSI.reshape(B * C * H, W), desc)
    patches = out3[:UNITS].reshape(B, N_PATCHES, C, PATCH, PATCH)
    return patches, sa_pad[:, 0, :N_PATCHES]


# speedup vs baseline: 1.3852x; 1.3852x over previous
import functools

import jax
import jax.numpy as jnp
from jax import lax
from jax.experimental import pallas as pl
from jax.experimental.pallas import tpu as pltpu
from jax.experimental.pallas import tpu_sc as plsc

N_PATCHES = 200
AH = AW = 128
H = W = 2048
C = 3
PATCH = 32
SY = H // AH
NC, NS = 2, 16
NW = NC * NS
UNITS = 2 * N_PATCHES * C
UPW = 38
UPAD = NW * UPW
BLKW = 256
KPAD = 256


def _topk_body(score_ref, p_ref, sa_ref, idx_ref):
    pos = (lax.broadcasted_iota(jnp.int32, (AH, AW), 0) * AW
           + lax.broadcasted_iota(jnp.int32, (AH, AW), 1))
    lane = lax.broadcasted_iota(jnp.int32, (KPAD,), 0)

    def step(j, s, pb, idxv, sav):
        m = jnp.max(s)
        chosen = jnp.min(jnp.where(s == m, pos, jnp.int32(1 << 30)))
        hit = pos == chosen
        pv = jnp.sum(jnp.where(hit, pb, jnp.float32(0.0)))
        s = jnp.where(hit, jnp.float32(-1e30), s)
        idxv = jnp.where(lane == j, chosen, idxv)
        sav = jnp.where(lane == j, pv, sav)
        return s, idxv, sav

    def body(j, st):
        s0, s1, i0, i1, a0, a1 = st
        s0, i0, a0 = step(j, s0, p_ref[0], i0, a0)
        s1, i1, a1 = step(j, s1, p_ref[1], i1, a1)
        return s0, s1, i0, i1, a0, a1

    z_i = jnp.zeros((KPAD,), jnp.int32)
    z_f = jnp.zeros((KPAD,), jnp.float32)
    _, _, i0, i1, a0, a1 = lax.fori_loop(
        0, N_PATCHES, body,
        (score_ref[0], score_ref[1], z_i, z_i, z_f, z_f))

    idx_ref[0, 0] = i0
    idx_ref[1, 0] = i1
    sa_ref[0, 0] = a0
    sa_ref[1, 0] = a1


def _topk_call(score, p):
    return pl.pallas_call(
        _topk_body,
        out_shape=[jax.ShapeDtypeStruct((2, 1, KPAD), jnp.float32),
                   jax.ShapeDtypeStruct((2, 1, KPAD), jnp.int32)],
    )(score, p)


@functools.cache
def _make_gather():
    mesh = plsc.VectorSubcoreMesh(core_axis_name="c", subcore_axis_name="s")

    @functools.partial(
        pl.kernel,
        mesh=mesh,
        out_type=jax.ShapeDtypeStruct((2, N_PATCHES, C, PATCH, PATCH),
                                      jnp.float32),
        compiler_params=pltpu.CompilerParams(use_tc_tiling_on_sc=True),
        scratch_types=[
            pltpu.VMEM((UPAD // 8, 128), jnp.int32),
            pltpu.VMEM((PATCH, BLKW), jnp.float32),
            pltpu.VMEM((PATCH, BLKW), jnp.float32),
            pltpu.VMEM((PATCH, PATCH), jnp.float32),
            pltpu.VMEM((PATCH, PATCH), jnp.float32),
            pltpu.SemaphoreType.DMA,
            pltpu.SemaphoreType.DMA,
            pltpu.SemaphoreType.DMA,
            pltpu.SemaphoreType.DMA,
        ],
    )
    def gather_k(wsi_hbm, desc_hbm, out_hbm, desc_v, buf0, buf1,
                 pbuf0, pbuf1, sr0, sr1, sw0, sw1):
        wid = lax.axis_index("s") * NC + lax.axis_index("c")
        pltpu.sync_copy(desc_hbm, desc_v)
        bufs = (buf0, buf1)
        pbufs = (pbuf0, pbuf1)
        srs = (sr0, sr1)
        sws = (sw0, sw1)

        def fields(t):
            u = t * NW + wid
            r = u // 8
            c0 = pl.multiple_of((u - r * 8) * 16, 16)
            v = desc_v[r, pl.ds(c0, 16)]
            return v[0], v[1], v[2], v[3], v[4], v[5]

        def start_read(t, buf, sem):
            row0, xa, _, _, _, _ = fields(t)
            row0 = pl.multiple_of(row0, 16)
            xa = pl.multiple_of(xa, 128)
            return pltpu.async_copy(
                wsi_hbm.at[pl.ds(row0, PATCH), pl.ds(xa, BLKW)], buf, sem)

        reads = [start_read(0, buf0, sr0), start_read(1, buf1, sr1)]
        writes = [None, None]
        for t in range(UPW):
            pipe = t % 2
            buf = bufs[pipe]
            pbuf = pbufs[pipe]
            reads[pipe].wait()
            if writes[pipe] is not None:
                writes[pipe].wait()
            _, _, xoff, ob, on, oc = fields(t)
            xoff = pl.multiple_of(xoff, 16)
            for r in range(PATCH):
                for h in range(2):
                    pbuf[r, pl.ds(h * 16, 16)] = (
                        buf[r, pl.ds(xoff + h * 16, 16)])
            writes[pipe] = pltpu.async_copy(
                pbuf, out_hbm.at[ob, on, oc], sws[pipe])
            if t + 2 < UPW:
                reads[pipe] = start_read(t + 2, buf, srs[pipe])
        writes[0].wait()
        writes[1].wait()

    return gather_k


def kernel(x_low, x_high, attention, WSI):
    B = attention.shape[0]
    flat = attention.reshape(B, -1)
    p = flat / jnp.sum(flat, axis=-1, keepdims=True)
    logp = jnp.log(p + 1e-12)
    u = jax.random.uniform(jax.random.key(42), flat.shape,
                           minval=1e-9, maxval=1.0)
    gumbel = -jnp.log(-jnp.log(u))
    score = logp + gumbel
    sa_pad, idx_pad = _topk_call(score.reshape(B, AH, AW),
                                 p.reshape(B, AH, AW))

    cell = idx_pad[:, 0, :N_PATCHES]
    ys = cell // AW
    xs = cell % AW
    y0 = jnp.minimum(ys * SY, H - PATCH)
    x0 = jnp.minimum(xs * SY, W - PATCH)
    xa = jnp.minimum((x0 // 128) * 128, W - BLKW)
    xoff = (x0 - xa)[:, :, None]
    xa = xa[:, :, None]
    cc = jnp.arange(C, dtype=jnp.int32)[None, None, :]
    bb = jnp.arange(B, dtype=jnp.int32)[:, None, None]
    nn = jnp.arange(N_PATCHES, dtype=jnp.int32)[None, :, None]
    row0 = (bb * C + cc) * H + y0[:, :, None]
    zz = jnp.zeros((B, N_PATCHES, C), jnp.int32)
    fields = jnp.stack(
        [row0, xa + zz, xoff + zz, bb + zz, nn + zz, cc + zz],
        axis=-1).reshape(UNITS, 6).astype(jnp.int32)
    fields = jnp.concatenate(
        [fields, jnp.broadcast_to(fields[:1], (UPAD - UNITS, 6))], axis=0)
    desc = jnp.pad(fields, ((0, 0), (0, 10))).reshape(UPAD // 8, 128)

    patches = _make_gather()(WSI.reshape(B * C * H, W), desc)
    return patches, sa_pad[:, 0, :N_PATCHES]


# speedup vs baseline: 1.9208x; 1.3867x over previous
import functools

import jax
import jax.numpy as jnp
from jax import lax
from jax.experimental import pallas as pl
from jax.experimental.pallas import tpu as pltpu
from jax.experimental.pallas import tpu_sc as plsc

N_PATCHES = 200
AH = AW = 128
H = W = 2048
C = 3
PATCH = 32
SY = H // AH
NC, NS = 2, 16
NW = NC * NS
UNITS = 2 * N_PATCHES * C
UPW = 38
UPAD = NW * UPW
BLKW = 256
KPAD = 256


NCH = 4
CHR = AH // NCH


def _topk_body(score_ref, p_ref, sa_ref, idx_ref):
    pos = (lax.broadcasted_iota(jnp.int32, (CHR, AW), 0) * AW
           + lax.broadcasted_iota(jnp.int32, (CHR, AW), 1))
    lane = lax.broadcasted_iota(jnp.int32, (KPAD,), 0)
    big = jnp.int32(1 << 30)
    neg = jnp.float32(-1e30)

    def step(j, chunks, pcs, cms, idxv, sav):
        m = jnp.maximum(jnp.maximum(cms[0], cms[1]),
                        jnp.maximum(cms[2], cms[3]))
        p0 = cms[0] == m
        p1 = jnp.logical_and(jnp.logical_not(p0), cms[1] == m)
        p01 = jnp.logical_or(p0, p1)
        p2 = jnp.logical_and(jnp.logical_not(p01), cms[2] == m)
        preds = (p0, p1, p2)
        sel = jnp.where(p0, chunks[0],
                        jnp.where(p1, chunks[1],
                                  jnp.where(p2, chunks[2], chunks[3])))
        psel = jnp.where(p0, pcs[0],
                         jnp.where(p1, pcs[1],
                                   jnp.where(p2, pcs[2], pcs[3])))
        base = jnp.where(p0, jnp.int32(0),
                         jnp.where(p1, jnp.int32(CHR * AW),
                                   jnp.where(p2, jnp.int32(2 * CHR * AW),
                                             jnp.int32(3 * CHR * AW))))
        local = jnp.min(jnp.where(sel == m, pos, big))
        hit = pos == local
        pv = jnp.sum(jnp.where(hit, psel, jnp.float32(0.0)))
        upd = jnp.where(hit, neg, sel)
        mx = jnp.max(upd)
        new_chunks = []
        new_cms = []
        for k in range(NCH):
            pk = preds[k] if k < 3 else jnp.logical_not(
                jnp.logical_or(p01, p2))
            new_chunks.append(jnp.where(pk, upd, chunks[k]))
            new_cms.append(jnp.where(pk, mx, cms[k]))
        idxv = jnp.where(lane == j, base + local, idxv)
        sav = jnp.where(lane == j, pv, sav)
        return tuple(new_chunks), tuple(new_cms), idxv, sav

    def body(j, st):
        c0, m0, i0, a0, c1, m1, i1, a1 = st
        c0, m0, i0, a0 = step(j, c0, _pc(0), m0, i0, a0)
        c1, m1, i1, a1 = step(j, c1, _pc(1), m1, i1, a1)
        return c0, m0, i0, a0, c1, m1, i1, a1

    def _pc(b):
        return tuple(p_ref[b, pl.ds(k * CHR, CHR), :] for k in range(NCH))

    z_i = jnp.zeros((KPAD,), jnp.int32)
    z_f = jnp.zeros((KPAD,), jnp.float32)

    def init(b):
        chunks = tuple(score_ref[b, pl.ds(k * CHR, CHR), :]
                       for k in range(NCH))
        cms = tuple(jnp.max(c) for c in chunks)
        return chunks, cms

    c0, m0 = init(0)
    c1, m1 = init(1)
    _, _, i0, a0, _, _, i1, a1 = lax.fori_loop(
        0, N_PATCHES, body, (c0, m0, z_i, z_f, c1, m1, z_i, z_f))

    idx_ref[0, 0] = i0
    idx_ref[1, 0] = i1
    sa_ref[0, 0] = a0
    sa_ref[1, 0] = a1


def _topk_call(score, p):
    return pl.pallas_call(
        _topk_body,
        out_shape=[jax.ShapeDtypeStruct((2, 1, KPAD), jnp.float32),
                   jax.ShapeDtypeStruct((2, 1, KPAD), jnp.int32)],
    )(score, p)


@functools.cache
def _make_gather():
    mesh = plsc.VectorSubcoreMesh(core_axis_name="c", subcore_axis_name="s")

    @functools.partial(
        pl.kernel,
        mesh=mesh,
        out_type=jax.ShapeDtypeStruct((2, N_PATCHES, C, PATCH, PATCH),
                                      jnp.float32),
        compiler_params=pltpu.CompilerParams(use_tc_tiling_on_sc=True),
        scratch_types=[
            pltpu.VMEM((UPAD // 8, 128), jnp.int32),
            pltpu.VMEM((PATCH, BLKW), jnp.float32),
            pltpu.VMEM((PATCH, BLKW), jnp.float32),
            pltpu.VMEM((PATCH, PATCH), jnp.float32),
            pltpu.VMEM((PATCH, PATCH), jnp.float32),
            pltpu.SemaphoreType.DMA,
            pltpu.SemaphoreType.DMA,
            pltpu.SemaphoreType.DMA,
            pltpu.SemaphoreType.DMA,
        ],
    )
    def gather_k(wsi_hbm, desc_hbm, out_hbm, desc_v, buf0, buf1,
                 pbuf0, pbuf1, sr0, sr1, sw0, sw1):
        wid = lax.axis_index("s") * NC + lax.axis_index("c")
        pltpu.sync_copy(desc_hbm, desc_v)
        bufs = (buf0, buf1)
        pbufs = (pbuf0, pbuf1)
        srs = (sr0, sr1)
        sws = (sw0, sw1)

        def fields(t):
            u = t * NW + wid
            r = u // 8
            c0 = pl.multiple_of((u - r * 8) * 16, 16)
            v = desc_v[r, pl.ds(c0, 16)]
            return v[0], v[1], v[2], v[3], v[4], v[5]

        def start_read(t, buf, sem):
            row0, xa, _, _, _, _ = fields(t)
            row0 = pl.multiple_of(row0, 16)
            xa = pl.multiple_of(xa, 128)
            return pltpu.async_copy(
                wsi_hbm.at[pl.ds(row0, PATCH), pl.ds(xa, BLKW)], buf, sem)

        reads = [start_read(0, buf0, sr0), start_read(1, buf1, sr1)]
        writes = [None, None]
        for t in range(UPW):
            pipe = t % 2
            buf = bufs[pipe]
            pbuf = pbufs[pipe]
            reads[pipe].wait()
            if writes[pipe] is not None:
                writes[pipe].wait()
            _, _, xoff, ob, on, oc = fields(t)
            xoff = pl.multiple_of(xoff, 16)
            for r in range(PATCH):
                for h in range(2):
                    pbuf[r, pl.ds(h * 16, 16)] = (
                        buf[r, pl.ds(xoff + h * 16, 16)])
            writes[pipe] = pltpu.async_copy(
                pbuf, out_hbm.at[ob, on, oc], sws[pipe])
            if t + 2 < UPW:
                reads[pipe] = start_read(t + 2, buf, srs[pipe])
        writes[0].wait()
        writes[1].wait()

    return gather_k


def kernel(x_low, x_high, attention, WSI):
    B = attention.shape[0]
    flat = attention.reshape(B, -1)
    p = flat / jnp.sum(flat, axis=-1, keepdims=True)
    logp = jnp.log(p + 1e-12)
    u = jax.random.uniform(jax.random.key(42), flat.shape,
                           minval=1e-9, maxval=1.0)
    gumbel = -jnp.log(-jnp.log(u))
    score = logp + gumbel
    sa_pad, idx_pad = _topk_call(score.reshape(B, AH, AW),
                                 p.reshape(B, AH, AW))

    cell = idx_pad[:, 0, :N_PATCHES]
    ys = cell // AW
    xs = cell % AW
    y0 = jnp.minimum(ys * SY, H - PATCH)
    x0 = jnp.minimum(xs * SY, W - PATCH)
    xa = jnp.minimum((x0 // 128) * 128, W - BLKW)
    xoff = (x0 - xa)[:, :, None]
    xa = xa[:, :, None]
    cc = jnp.arange(C, dtype=jnp.int32)[None, None, :]
    bb = jnp.arange(B, dtype=jnp.int32)[:, None, None]
    nn = jnp.arange(N_PATCHES, dtype=jnp.int32)[None, :, None]
    row0 = (bb * C + cc) * H + y0[:, :, None]
    zz = jnp.zeros((B, N_PATCHES, C), jnp.int32)
    fields = jnp.stack(
        [row0, xa + zz, xoff + zz, bb + zz, nn + zz, cc + zz],
        axis=-1).reshape(UNITS, 6).astype(jnp.int32)
    fields = jnp.concatenate(
        [fields, jnp.broadcast_to(fields[:1], (UPAD - UNITS, 6))], axis=0)
    desc = jnp.pad(fields, ((0, 0), (0, 10))).reshape(UPAD // 8, 128)

    patches = _make_gather()(WSI.reshape(B * C * H, W), desc)
    return patches, sa_pad[:, 0, :N_PATCHES]
